# baseline (speedup 1.0000x reference)
import functools

import jax
import jax.numpy as jnp
from jax import lax
from jax.experimental import pallas as pl
from jax.experimental.pallas import tpu as pltpu
from jax.experimental.pallas import tpu_sc as plsc

_DIM = 64
_BW = 128


def _gather_all(start_t, path_t, end_t, te, pe):
    mp_total, b_total = start_t.shape
    info = plsc.get_sparse_core_info()
    nw = info.num_cores * info.num_subcores
    nj = b_total // _BW
    units = mp_total * nj
    units_per_w = units // nw
    tile_cols = 8 * _BW

    mesh = plsc.VectorSubcoreMesh(core_axis_name="c", subcore_axis_name="s")

    @functools.partial(
        pl.kernel,
        out_type=jax.ShapeDtypeStruct((mp_total, 3, _DIM // 8, nj, 8, _BW), jnp.float32),
        mesh=mesh,
        scratch_types=[
            pltpu.VMEM((_BW,), jnp.int32),
            pltpu.VMEM((_BW, _DIM), jnp.float32),
            pltpu.VMEM((_DIM // 8, 8, _BW), jnp.float32),
            pltpu.SemaphoreType.DMA,
        ],
        compiler_params=pltpu.CompilerParams(
            use_tc_tiling_on_sc=False, needs_layout_passes=False
        ),
    )
    def k(start_h, path_h, end_h, te_h, pe_h, out_h, idx_v, rows_v, trans_v, sem):
        wid = lax.axis_index("s") * info.num_cores + lax.axis_index("c")
        ubase = wid * units_per_w
        lane = lax.iota(jnp.int32, 16)

        def do_slot(mp, j, idx_h, tab_h, slot):
            pltpu.sync_copy(idx_h.at[mp, pl.ds(j * _BW, _BW)], idx_v)
            pltpu.async_copy(tab_h.at[idx_v], rows_v, sem).wait()

            def dloop(d, carry):
                drow = d >> 3
                din = d & 7
                for bg in range(_BW // 16):
                    vals = plsc.load_gather(
                        rows_v, [bg * 16 + lane, jnp.full((16,), d, jnp.int32)]
                    )
                    trans_v[drow, din, pl.ds(bg * 16, 16)] = vals
                return carry

            lax.fori_loop(0, _DIM, dloop, 0)
            pltpu.sync_copy(trans_v, out_h.at[mp, slot, :, j, :, :])

        def unit_body(t, carry):
            u = ubase + t
            mp = u >> 3
            j = u & 7
            do_slot(mp, j, start_h, te_h, 0)
            do_slot(mp, j, path_h, pe_h, 1)
            do_slot(mp, j, end_h, te_h, 2)
            return carry

        lax.fori_loop(0, units_per_w, unit_body, 0)

    return k(start_t, path_t, end_t, te, pe)


def kernel(start, path, end, te, pe):
    b, mp, _ = start.shape
    start_t = start.reshape(b, mp).T
    path_t = path.reshape(b, mp).T
    end_t = end.reshape(b, mp).T
    out6 = _gather_all(start_t, path_t, end_t, te, pe)
    return out6.transpose(3, 5, 0, 1, 2, 4).reshape(b, mp, 3, _DIM)


# speedup vs baseline: 1.1456x; 1.1456x over previous
import functools

import jax
import jax.numpy as jnp
from jax import lax
from jax.experimental import pallas as pl
from jax.experimental.pallas import tpu as pltpu
from jax.experimental.pallas import tpu_sc as plsc

_DIM = 64
_BW = 128
_STEPS_PER_IT = 6


def _gather_all(start_t, path_t, end_t, te, pe):
    mp_total, b_total = start_t.shape
    info = plsc.get_sparse_core_info()
    nw = info.num_cores * info.num_subcores
    nj = b_total // _BW
    units_per_w = mp_total * nj // nw
    steps_per_w = units_per_w * 3
    n_iters = steps_per_w // _STEPS_PER_IT

    mesh = plsc.VectorSubcoreMesh(core_axis_name="c", subcore_axis_name="s")

    @functools.partial(
        pl.kernel,
        out_type=jax.ShapeDtypeStruct((mp_total, 3, _DIM // 8, nj, 8, _BW), jnp.float32),
        mesh=mesh,
        scratch_types=[
            pltpu.VMEM((2, _BW), jnp.int32),
            pltpu.VMEM((2, _BW, _DIM), jnp.float32),
            pltpu.VMEM((2, _DIM // 8, 8, _BW), jnp.float32),
            pltpu.SemaphoreType.DMA,
            pltpu.SemaphoreType.DMA,
            pltpu.SemaphoreType.DMA,
            pltpu.SemaphoreType.DMA,
        ],
        compiler_params=pltpu.CompilerParams(
            use_tc_tiling_on_sc=False, needs_layout_passes=False
        ),
    )
    def k(start_h, path_h, end_h, te_h, pe_h, out_h, idx_v, rows_v, trans_v,
          gsem0, gsem1, wsem0, wsem1):
        wid = lax.axis_index("s") * info.num_cores + lax.axis_index("c")
        ubase = wid * units_per_w
        lane = lax.iota(jnp.int32, 16)
        gsems = (gsem0, gsem1)
        wsems = (wsem0, wsem1)

        def refs_for(s):
            return ((start_h, te_h), (path_h, pe_h), (end_h, te_h))[s % 3]

        def launch(t, s):
            p = s % 2
            u = ubase + t * (_STEPS_PER_IT // 3) + s // 3
            mp = u >> 3
            j = u & 7
            idx_h, tab_h = refs_for(s)
            pltpu.sync_copy(idx_h.at[mp, pl.ds(j * _BW, _BW)], idx_v.at[p])
            return pltpu.async_copy(tab_h.at[idx_v.at[p]], rows_v.at[p], gsems[p])

        def finish(t, s, ghandle):
            p = s % 2
            u = ubase + t * (_STEPS_PER_IT // 3) + s // 3
            mp = u >> 3
            j = u & 7
            slot = s % 3
            ghandle.wait()

            def drow_body(drow, carry):
                for din in range(8):
                    d_splat = jnp.full((16,), drow * 8 + din, jnp.int32)
                    for bg in range(_BW // 16):
                        vals = plsc.load_gather(
                            rows_v.at[p], [bg * 16 + lane, d_splat]
                        )
                        trans_v[p, drow, din, pl.ds(bg * 16, 16)] = vals
                return carry

            lax.fori_loop(0, _DIM // 8, drow_body, 0)
            return pltpu.async_copy(
                trans_v.at[p], out_h.at[mp, slot, :, j, :, :], wsems[p]
            )

        def iter_body(t, carry):
            handles = [None, None]
            whandles = [None, None]
            handles[0] = launch(t, 0)
            for s in range(_STEPS_PER_IT - 1):
                handles[(s + 1) % 2] = launch(t, s + 1)
                if whandles[s % 2] is not None:
                    whandles[s % 2].wait()
                whandles[s % 2] = finish(t, s, handles[s % 2])
            s_last = _STEPS_PER_IT - 1
            if whandles[s_last % 2] is not None:
                whandles[s_last % 2].wait()
            whandles[s_last % 2] = finish(t, s_last, handles[s_last % 2])
            whandles[0].wait()
            whandles[1].wait()
            return carry

        lax.fori_loop(0, n_iters, iter_body, 0)

    return k(start_t, path_t, end_t, te, pe)


def kernel(start, path, end, te, pe):
    b, mp, _ = start.shape
    start_t = start.reshape(b, mp).T
    path_t = path.reshape(b, mp).T
    end_t = end.reshape(b, mp).T
    out6 = _gather_all(start_t, path_t, end_t, te, pe)
    return out6.transpose(3, 5, 0, 1, 2, 4).reshape(b, mp, 3, _DIM)


# speedup vs baseline: 4.0756x; 3.5578x over previous
import functools

import jax
import jax.numpy as jnp
from jax import lax
from jax.experimental import pallas as pl
from jax.experimental.pallas import tpu as pltpu
from jax.experimental.pallas import tpu_sc as plsc

_DIM = 64
_BW = 128
_STEPS_PER_IT = 6


def _gather_all(start_t, path_t, end_t, te, pe):
    mp_total, b_total = start_t.shape
    info = plsc.get_sparse_core_info()
    nw = info.num_cores * info.num_subcores
    nj = b_total // _BW
    units_per_w = mp_total * nj // nw
    steps_per_w = units_per_w * 3
    n_iters = steps_per_w // _STEPS_PER_IT

    mesh = plsc.VectorSubcoreMesh(core_axis_name="c", subcore_axis_name="s")

    @functools.partial(
        pl.kernel,
        out_type=jax.ShapeDtypeStruct(
            (mp_total, 3, _DIM // 8, nj, 8, _BW), jnp.float32
        ),
        mesh=mesh,
        scratch_types=[
            pltpu.VMEM((_BW,), jnp.int32),
            pltpu.VMEM((_BW,), jnp.int32),
            pltpu.VMEM((_BW, _DIM), jnp.float32),
            pltpu.VMEM((_BW, _DIM), jnp.float32),
            pltpu.VMEM((_DIM, _BW), jnp.float32),
            pltpu.VMEM((_DIM, _BW), jnp.float32),
            pltpu.SemaphoreType.DMA,
            pltpu.SemaphoreType.DMA,
            pltpu.SemaphoreType.DMA,
            pltpu.SemaphoreType.DMA,
        ],
        compiler_params=pltpu.CompilerParams(
            use_tc_tiling_on_sc=False, needs_layout_passes=False
        ),
    )
    def k(start_h, path_h, end_h, te_h, pe_h, out_h,
          idx_a, idx_b, rows_a, rows_b, trans_a, trans_b,
          gsem0, gsem1, wsem0, wsem1):
        wid = lax.axis_index("s") * info.num_cores + lax.axis_index("c")
        ubase = wid * units_per_w
        lane = lax.iota(jnp.int32, 16)
        bvecs = [lane + bg * 16 for bg in range(_BW // 16)]
        idxs = (idx_a, idx_b)
        rows = (rows_a, rows_b)
        trans = (trans_a, trans_b)
        gsems = (gsem0, gsem1)
        wsems = (wsem0, wsem1)

        def refs_for(s):
            return ((start_h, te_h), (path_h, pe_h), (end_h, te_h))[s % 3]

        def decode(t, s):
            u = ubase + t * (_STEPS_PER_IT // 3) + s // 3
            return u >> 3, u & 7

        def launch(t, s):
            p = s % 2
            mp, j = decode(t, s)
            idx_h, tab_h = refs_for(s)
            pltpu.sync_copy(idx_h.at[mp, pl.ds(j * _BW, _BW)], idxs[p])
            return pltpu.async_copy(tab_h.at[idxs[p]], rows[p], gsems[p])

        def finish(t, s, ghandle):
            p = s % 2
            mp, j = decode(t, s)
            slot = s % 3
            rows_p = rows[p]
            trans_p = trans[p]
            ghandle.wait()

            @plsc.parallel_loop(0, _DIM, unroll=4)
            def wave_body(i):
                dvec = (i & ~15) + ((lane + i) & 15)
                for bg in range(_BW // 16):
                    vals = plsc.load_gather(rows_p, [bvecs[bg], dvec])
                    plsc.store_scatter(trans_p, [dvec, bvecs[bg]], vals)

            handles = []
            for i in range(_DIM // 8):
                handles.append(
                    pltpu.async_copy(
                        trans_p.at[pl.ds(i * 8, 8), :],
                        out_h.at[mp, slot, i, j, :, :],
                        wsems[p],
                    )
                )
            return handles

        def iter_body(t, carry):
            handles = [None, None]
            whandles = [None, None]
            handles[0] = launch(t, 0)
            for s in range(_STEPS_PER_IT - 1):
                handles[(s + 1) % 2] = launch(t, s + 1)
                if whandles[s % 2] is not None:
                    for h in whandles[s % 2]:
                        h.wait()
                whandles[s % 2] = finish(t, s, handles[s % 2])
            s_last = _STEPS_PER_IT - 1
            if whandles[s_last % 2] is not None:
                for h in whandles[s_last % 2]:
                    h.wait()
            whandles[s_last % 2] = finish(t, s_last, handles[s_last % 2])
            for h in whandles[0]:
                h.wait()
            for h in whandles[1]:
                h.wait()
            return carry

        lax.fori_loop(0, n_iters, iter_body, 0)

    return k(start_t, path_t, end_t, te, pe)


def kernel(start, path, end, te, pe):
    b, mp, _ = start.shape
    start_t = start.reshape(b, mp).T
    path_t = path.reshape(b, mp).T
    end_t = end.reshape(b, mp).T
    out6 = _gather_all(start_t, path_t, end_t, te, pe)
    return out6.transpose(3, 5, 0, 1, 2, 4).reshape(b, mp, 3, _DIM)


# speedup vs baseline: 4.7184x; 1.1577x over previous
import functools

import jax
import jax.numpy as jnp
from jax import lax
from jax.experimental import pallas as pl
from jax.experimental.pallas import tpu as pltpu
from jax.experimental.pallas import tpu_sc as plsc

_DIM = 64
_BW = 128
_UNITS_PER_IT = 2


def _gather_all(idx3, te, pe):
    _, mp_total, b_total = idx3.shape
    info = plsc.get_sparse_core_info()
    nw = info.num_cores * info.num_subcores
    nj = b_total // _BW
    units_per_w = mp_total * nj // nw
    n_iters = units_per_w // _UNITS_PER_IT

    mesh = plsc.VectorSubcoreMesh(core_axis_name="c", subcore_axis_name="s")

    @functools.partial(
        pl.kernel,
        out_type=jax.ShapeDtypeStruct(
            (mp_total, 3, _DIM // 8, nj, 8, _BW), jnp.float32
        ),
        mesh=mesh,
        scratch_types=[
            pltpu.VMEM((3, _BW), jnp.int32),
            pltpu.VMEM((3, _BW), jnp.int32),
            pltpu.VMEM((3, _BW, _DIM), jnp.float32),
            pltpu.VMEM((3, _BW, _DIM), jnp.float32),
            pltpu.VMEM((3, _DIM, _BW), jnp.float32),
            pltpu.VMEM((3, _DIM, _BW), jnp.float32),
            pltpu.SemaphoreType.DMA,
            pltpu.SemaphoreType.DMA,
            pltpu.SemaphoreType.DMA,
            pltpu.SemaphoreType.DMA,
        ],
        compiler_params=pltpu.CompilerParams(
            use_tc_tiling_on_sc=False, needs_layout_passes=False
        ),
    )
    def k(idx3_h, te_h, pe_h, out_h,
          idx_a, idx_b, rows_a, rows_b, trans_a, trans_b,
          gsem0, gsem1, wsem0, wsem1):
        wid = lax.axis_index("s") * info.num_cores + lax.axis_index("c")
        ubase = wid * units_per_w
        lane = lax.iota(jnp.int32, 16)
        bvecs = [lane + bg * 16 for bg in range(_BW // 16)]
        idxs = (idx_a, idx_b)
        rows = (rows_a, rows_b)
        trans = (trans_a, trans_b)
        gsems = (gsem0, gsem1)
        wsems = (wsem0, wsem1)
        tabs = (te_h, pe_h, te_h)

        def decode(t, i):
            u = ubase + t * _UNITS_PER_IT + i
            return u >> 3, u & 7

        def launch(t, i):
            p = i % 2
            mp, j = decode(t, i)
            pltpu.sync_copy(idx3_h.at[:, mp, pl.ds(j * _BW, _BW)], idxs[p])
            return [
                pltpu.async_copy(
                    tabs[s].at[idxs[p].at[s]], rows[p].at[s], gsems[p]
                )
                for s in range(3)
            ]

        def finish(t, i, ghandles):
            p = i % 2
            mp, j = decode(t, i)
            for h in ghandles:
                h.wait()
            whandles = []
            for s in range(3):
                rows_p = rows[p].at[s]
                trans_p = trans[p].at[s]

                @plsc.parallel_loop(0, _DIM, unroll=4)
                def wave_body(w):
                    dvec = (w & ~15) + ((lane + w) & 15)
                    for bg in range(_BW // 16):
                        vals = plsc.load_gather(rows_p, [bvecs[bg], dvec])
                        plsc.store_scatter(trans_p, [dvec, bvecs[bg]], vals)

                for ti in range(_DIM // 8):
                    whandles.append(
                        pltpu.async_copy(
                            trans_p.at[pl.ds(ti * 8, 8), :],
                            out_h.at[mp, s, ti, j, :, :],
                            wsems[p],
                        )
                    )
            return whandles

        def iter_body(t, carry):
            ghandles = [None, None]
            whandles = [None, None]
            ghandles[0] = launch(t, 0)
            for i in range(_UNITS_PER_IT - 1):
                ghandles[(i + 1) % 2] = launch(t, i + 1)
                if whandles[i % 2] is not None:
                    for h in whandles[i % 2]:
                        h.wait()
                whandles[i % 2] = finish(t, i, ghandles[i % 2])
            i_last = _UNITS_PER_IT - 1
            if whandles[i_last % 2] is not None:
                for h in whandles[i_last % 2]:
                    h.wait()
            whandles[i_last % 2] = finish(t, i_last, ghandles[i_last % 2])
            for h in whandles[0]:
                h.wait()
            for h in whandles[1]:
                h.wait()
            return carry

        lax.fori_loop(0, n_iters, iter_body, 0)

    return k(idx3, te, pe)


def kernel(start, path, end, te, pe):
    b, mp, _ = start.shape
    idx3 = jnp.stack(
        [start.reshape(b, mp).T, path.reshape(b, mp).T, end.reshape(b, mp).T]
    )
    out6 = _gather_all(idx3, te, pe)
    return out6.transpose(3, 5, 0, 1, 2, 4).reshape(b, mp, 3, _DIM)


# speedup vs baseline: 5.2268x; 1.1078x over previous
import functools

import jax
import jax.numpy as jnp
from jax import lax
from jax.experimental import pallas as pl
from jax.experimental.pallas import tpu as pltpu
from jax.experimental.pallas import tpu_sc as plsc

_DIM = 64
_BW = 128
_UNITS_PER_IT = 10


def _gather_all(idx3, te, pe):
    _, mp_total, b_total = idx3.shape
    info = plsc.get_sparse_core_info()
    nw = info.num_cores * info.num_subcores
    nj = b_total // _BW
    units_per_w = mp_total * nj // nw
    n_iters = units_per_w // _UNITS_PER_IT

    mesh = plsc.VectorSubcoreMesh(core_axis_name="c", subcore_axis_name="s")

    @functools.partial(
        pl.kernel,
        out_type=jax.ShapeDtypeStruct(
            (mp_total, 3, _DIM // 8, nj, 8, _BW), jnp.float32
        ),
        mesh=mesh,
        scratch_types=[
            pltpu.VMEM((3, _BW), jnp.int32),
            pltpu.VMEM((3, _BW), jnp.int32),
            pltpu.VMEM((3, _BW, _DIM), jnp.float32),
            pltpu.VMEM((3, _BW, _DIM), jnp.float32),
            pltpu.VMEM((3, _DIM, _BW), jnp.float32),
            pltpu.VMEM((3, _DIM, _BW), jnp.float32),
            pltpu.SemaphoreType.DMA,
            pltpu.SemaphoreType.DMA,
            pltpu.SemaphoreType.DMA,
            pltpu.SemaphoreType.DMA,
            pltpu.SemaphoreType.DMA,
            pltpu.SemaphoreType.DMA,
        ],
        compiler_params=pltpu.CompilerParams(
            use_tc_tiling_on_sc=False, needs_layout_passes=False
        ),
    )
    def k(idx3_h, te_h, pe_h, out_h,
          idx_a, idx_b, rows_a, rows_b, trans_a, trans_b,
          gsem0, gsem1, wsem0, wsem1, isem0, isem1):
        wid = lax.axis_index("s") * info.num_cores + lax.axis_index("c")
        ubase = wid * units_per_w
        lane = lax.iota(jnp.int32, 16)
        bvecs = [lane + bg * 16 for bg in range(_BW // 16)]
        idxs = (idx_a, idx_b)
        rows = (rows_a, rows_b)
        trans = (trans_a, trans_b)
        gsems = (gsem0, gsem1)
        wsems = (wsem0, wsem1)
        isems = (isem0, isem1)
        tabs = (te_h, pe_h, te_h)

        def decode(t, i):
            u = ubase + t * _UNITS_PER_IT + i
            return u >> 3, u & 7

        def idx_start(t, i):
            p = i % 2
            mp, j = decode(t, i)
            return pltpu.async_copy(
                idx3_h.at[:, mp, pl.ds(j * _BW, _BW)], idxs[p], isems[p]
            )

        def gfire(t, i, ihandle):
            p = i % 2
            ihandle.wait()
            return [
                pltpu.async_copy(
                    tabs[s].at[idxs[p].at[s]], rows[p].at[s], gsems[p]
                )
                for s in range(3)
            ]

        def finish(t, i):
            p = i % 2
            mp, j = decode(t, i)
            whandles = []
            for s in range(3):
                rows_p = rows[p].at[s]
                trans_p = trans[p].at[s]

                @plsc.parallel_loop(0, _DIM, unroll=4)
                def wave_body(w):
                    dvec = (w & ~15) + ((lane + w) & 15)
                    for bg in range(_BW // 16):
                        vals = plsc.load_gather(rows_p, [bvecs[bg], dvec])
                        plsc.store_scatter(trans_p, [dvec, bvecs[bg]], vals)

                for ti in range(_DIM // 8):
                    whandles.append(
                        pltpu.async_copy(
                            trans_p.at[pl.ds(ti * 8, 8), :],
                            out_h.at[mp, s, ti, j, :, :],
                            wsems[p],
                        )
                    )
            return whandles

        def iter_body(t, carry):
            ih = [None, None]
            gh = [None, None]
            wh = [None, None]
            ih[0] = idx_start(t, 0)
            ih[1] = idx_start(t, 1)
            gh[0] = gfire(t, 0, ih[0])
            for i in range(_UNITS_PER_IT):
                p = i % 2
                if i + 1 < _UNITS_PER_IT:
                    gh[(i + 1) % 2] = gfire(t, i + 1, ih[(i + 1) % 2])
                if wh[p] is not None:
                    for h in wh[p]:
                        h.wait()
                for h in gh[p]:
                    h.wait()
                if i + 2 < _UNITS_PER_IT:
                    ih[p] = idx_start(t, i + 2)
                wh[p] = finish(t, i)
            for h in wh[0]:
                h.wait()
            for h in wh[1]:
                h.wait()
            return carry

        lax.fori_loop(0, n_iters, iter_body, 0)

    return k(idx3, te, pe)


def kernel(start, path, end, te, pe):
    b, mp, _ = start.shape
    idx3 = jnp.stack(
        [start.reshape(b, mp).T, path.reshape(b, mp).T, end.reshape(b, mp).T]
    )
    out6 = _gather_all(idx3, te, pe)
    return out6.transpose(3, 5, 0, 1, 2, 4).reshape(b, mp, 3, _DIM)
